# baseline (device time: 22052 ns/iter reference)
import jax
import jax.numpy as jnp
from jax import lax
from jax.experimental import pallas as pl
from jax.experimental.pallas import tpu as pltpu

_BLOCK_M = 512
_EPS = 1e-5


def kernel(x, dy, gamma):
    m, d = x.shape
    n_blocks = m // _BLOCK_M

    def body(x_ref, dy_ref, gamma_ref, out_ref, acc_ref, recv_ref,
             send_sem, recv_sem):
        i = pl.program_id(0)

        my_x = lax.axis_index("x")
        my_y = lax.axis_index("y")
        my_z = lax.axis_index("z")
        peer = (my_x, 1 - my_y, my_z)

        @pl.when(i == 0)
        def _():
            barrier_sem = pltpu.get_barrier_semaphore()
            pl.semaphore_signal(
                barrier_sem, inc=1, device_id=peer,
                device_id_type=pl.DeviceIdType.MESH,
            )

        xb = x_ref[:, :]
        dyb = dy_ref[:, :]
        mb = xb.shape[0]

        xsq = xb * xb
        xdy = xb * dyb
        ones_d = jnp.ones((d, 1), jnp.float32)
        s1 = jnp.dot(xb, ones_d, preferred_element_type=jnp.float32)
        s2 = jnp.dot(xsq, ones_d, preferred_element_type=jnp.float32)
        mu = s1 * (1.0 / d)
        var = s2 * (1.0 / d) - mu * mu
        rstd = lax.rsqrt(var + _EPS)

        def colsum(v, mat):
            return lax.dot_general(
                v, mat, (((0,), (0,)), ((), ())),
                preferred_element_type=jnp.float32,
            )

        ones_m = jnp.ones((mb, 1), jnp.float32)
        p = colsum(rstd, xdy)
        q = colsum(jnp.concatenate([mu * rstd, ones_m], axis=1), dyb)
        dgamma = p - q[0:1, :]
        dbeta = q[1:2, :]

        @pl.when(i == 0)
        def _():
            acc_ref[0:1, :] = dgamma
            acc_ref[1:2, :] = dbeta

        @pl.when(i > 0)
        def _():
            acc_ref[0:1, :] = acc_ref[0:1, :] + dgamma
            acc_ref[1:2, :] = acc_ref[1:2, :] + dbeta

        @pl.when(i == n_blocks - 1)
        def _():
            barrier_sem = pltpu.get_barrier_semaphore()
            pl.semaphore_wait(barrier_sem, 1)

            rdma = pltpu.make_async_remote_copy(
                src_ref=acc_ref,
                dst_ref=recv_ref,
                send_sem=send_sem,
                recv_sem=recv_sem,
                device_id=peer,
                device_id_type=pl.DeviceIdType.MESH,
            )
            rdma.start()
            rdma.wait()
            out_ref[:, :] = acc_ref[:, :] + recv_ref[:, :]

    return pl.pallas_call(
        body,
        grid=(n_blocks,),
        out_shape=jax.ShapeDtypeStruct((2, d), jnp.float32),
        in_specs=[
            pl.BlockSpec((_BLOCK_M, d), lambda i: (i, 0)),
            pl.BlockSpec((_BLOCK_M, d), lambda i: (i, 0)),
            pl.BlockSpec(memory_space=pl.ANY),
        ],
        out_specs=pl.BlockSpec((2, d), lambda i: (0, 0)),
        scratch_shapes=[
            pltpu.VMEM((2, d), jnp.float32),
            pltpu.VMEM((2, d), jnp.float32),
            pltpu.SemaphoreType.DMA,
            pltpu.SemaphoreType.DMA,
        ],
        compiler_params=pltpu.CompilerParams(collective_id=0),
    )(x, dy, gamma)


# device time: 21885 ns/iter; 1.0076x vs baseline; 1.0076x over previous
import jax
import jax.numpy as jnp
from jax import lax
from jax.experimental import pallas as pl
from jax.experimental.pallas import tpu as pltpu

_BLOCK_M = 512
_EPS = 1e-5


def kernel(x, dy, gamma):
    m, d = x.shape
    n_blocks = m // _BLOCK_M

    def body(x_ref, dy_ref, gamma_ref, out_ref, acc_ref, recv_ref,
             send_sem, recv_sem):
        i = pl.program_id(0)

        my_x = lax.axis_index("x")
        my_y = lax.axis_index("y")
        my_z = lax.axis_index("z")
        peer = (my_x, 1 - my_y, my_z)

        @pl.when(i == 0)
        def _():
            barrier_sem = pltpu.get_barrier_semaphore()
            pl.semaphore_signal(
                barrier_sem, inc=1, device_id=peer,
                device_id_type=pl.DeviceIdType.MESH,
            )

        xb = x_ref[:, :].astype(jnp.bfloat16)
        dyb = dy_ref[:, :].astype(jnp.bfloat16)
        mb = xb.shape[0]

        xsq = xb * xb
        xdy = xb * dyb
        ones_d = jnp.ones((d, 1), jnp.bfloat16)
        s1 = jnp.dot(xb, ones_d, preferred_element_type=jnp.float32)
        s2 = jnp.dot(xsq, ones_d, preferred_element_type=jnp.float32)
        mu = s1 * (1.0 / d)
        var = s2 * (1.0 / d) - mu * mu
        rstd = lax.rsqrt(var + _EPS)

        def colsum(v, mat):
            return lax.dot_general(
                v.astype(jnp.bfloat16), mat, (((0,), (0,)), ((), ())),
                preferred_element_type=jnp.float32,
            )

        ones_m = jnp.ones((mb, 1), jnp.float32)
        p = colsum(rstd, xdy)
        q = colsum(jnp.concatenate([mu * rstd, ones_m], axis=1), dyb)
        dgamma = p - q[0:1, :]
        dbeta = q[1:2, :]

        @pl.when(i == 0)
        def _():
            acc_ref[0:1, :] = dgamma
            acc_ref[1:2, :] = dbeta

        @pl.when(i > 0)
        def _():
            acc_ref[0:1, :] = acc_ref[0:1, :] + dgamma
            acc_ref[1:2, :] = acc_ref[1:2, :] + dbeta

        @pl.when(i == n_blocks - 1)
        def _():
            barrier_sem = pltpu.get_barrier_semaphore()
            pl.semaphore_wait(barrier_sem, 1)

            rdma = pltpu.make_async_remote_copy(
                src_ref=acc_ref,
                dst_ref=recv_ref,
                send_sem=send_sem,
                recv_sem=recv_sem,
                device_id=peer,
                device_id_type=pl.DeviceIdType.MESH,
            )
            rdma.start()
            rdma.wait()
            out_ref[:, :] = acc_ref[:, :] + recv_ref[:, :]

    return pl.pallas_call(
        body,
        grid=(n_blocks,),
        out_shape=jax.ShapeDtypeStruct((2, d), jnp.float32),
        in_specs=[
            pl.BlockSpec((_BLOCK_M, d), lambda i: (i, 0)),
            pl.BlockSpec((_BLOCK_M, d), lambda i: (i, 0)),
            pl.BlockSpec(memory_space=pl.ANY),
        ],
        out_specs=pl.BlockSpec((2, d), lambda i: (0, 0)),
        scratch_shapes=[
            pltpu.VMEM((2, d), jnp.float32),
            pltpu.VMEM((2, d), jnp.float32),
            pltpu.SemaphoreType.DMA,
            pltpu.SemaphoreType.DMA,
        ],
        compiler_params=pltpu.CompilerParams(collective_id=0),
    )(x, dy, gamma)


# device time: 20061 ns/iter; 1.0992x vs baseline; 1.0909x over previous
import jax
import jax.numpy as jnp
from jax import lax
from jax.experimental import pallas as pl
from jax.experimental.pallas import tpu as pltpu

_BLOCK_M = 512
_EPS = 1e-5


def kernel(x, dy, gamma):
    m, d = x.shape
    n_blocks = m // _BLOCK_M

    def body(x_ref, dy_ref, gamma_ref, out_ref, acc_ref, recv_ref,
             send_sem, recv_sem):
        i = pl.program_id(0)

        my_x = lax.axis_index("x")
        my_y = lax.axis_index("y")
        my_z = lax.axis_index("z")
        peer = (my_x, 1 - my_y, my_z)

        @pl.when(i == 0)
        def _():
            barrier_sem = pltpu.get_barrier_semaphore()
            pl.semaphore_signal(
                barrier_sem, inc=1, device_id=peer,
                device_id_type=pl.DeviceIdType.MESH,
            )

        xb = x_ref[:, :]
        dyb = dy_ref[:, :]
        s1 = jnp.sum(xb, axis=1, keepdims=True)
        s2 = jnp.sum(xb * xb, axis=1, keepdims=True)
        mu = s1 * (1.0 / d)
        var = s2 * (1.0 / d) - mu * mu
        rstd = lax.rsqrt(var + _EPS)
        xhat = xb * rstd + (-mu * rstd)
        dgamma = jnp.sum(dyb * xhat, axis=0, keepdims=True)
        dbeta = jnp.sum(dyb, axis=0, keepdims=True)

        @pl.when(i == 0)
        def _():
            acc_ref[0:1, :] = dgamma
            acc_ref[1:2, :] = dbeta

        @pl.when(i > 0)
        def _():
            acc_ref[0:1, :] = acc_ref[0:1, :] + dgamma
            acc_ref[1:2, :] = acc_ref[1:2, :] + dbeta

        @pl.when(i == n_blocks - 1)
        def _():
            barrier_sem = pltpu.get_barrier_semaphore()
            pl.semaphore_wait(barrier_sem, 1)

            rdma = pltpu.make_async_remote_copy(
                src_ref=acc_ref,
                dst_ref=recv_ref,
                send_sem=send_sem,
                recv_sem=recv_sem,
                device_id=peer,
                device_id_type=pl.DeviceIdType.MESH,
            )
            rdma.start()
            rdma.wait()
            out_ref[:, :] = acc_ref[:, :] + recv_ref[:, :]

    return pl.pallas_call(
        body,
        grid=(n_blocks,),
        out_shape=jax.ShapeDtypeStruct((2, d), jnp.float32),
        in_specs=[
            pl.BlockSpec((_BLOCK_M, d), lambda i: (i, 0)),
            pl.BlockSpec((_BLOCK_M, d), lambda i: (i, 0)),
            pl.BlockSpec(memory_space=pl.ANY),
        ],
        out_specs=pl.BlockSpec((2, d), lambda i: (0, 0)),
        scratch_shapes=[
            pltpu.VMEM((2, d), jnp.float32),
            pltpu.VMEM((2, d), jnp.float32),
            pltpu.SemaphoreType.DMA,
            pltpu.SemaphoreType.DMA,
        ],
        compiler_params=pltpu.CompilerParams(collective_id=0),
    )(x, dy, gamma)
